# baseline (speedup 1.0000x reference)
import functools

import jax
import jax.numpy as jnp
from jax import lax
from jax.experimental import pallas as pl
from jax.experimental.pallas import tpu as pltpu
from jax.experimental.pallas import tpu_sc as plsc

D = 1024
NC, NS = 2, 16
NW = NC * NS
LANES = 16
CH = 32


def _emb_kernel(b_total, t_len):
    b_per_w = b_total // NW
    n_chunks = b_per_w // CH
    mesh = plsc.VectorSubcoreMesh(
        core_axis_name="c", subcore_axis_name="s", num_cores=NC,
        num_subcores=NS)

    @functools.partial(
        pl.kernel,
        out_type=jax.ShapeDtypeStruct((b_total, D), jnp.float32),
        mesh=mesh,
        scratch_types=[
            pltpu.VMEM((b_per_w,), jnp.int32),
            pltpu.VMEM((CH, D), jnp.float32),
            pltpu.VMEM((CH, D), jnp.float32),
            pltpu.SemaphoreType.DMA,
        ],
    )
    def k(idx_hbm, tok_hbm, pos_hbm, out_hbm, idx_v, tok_v, pos_v, sem):
        wid = lax.axis_index("s") * NC + lax.axis_index("c")
        base = wid * b_per_w
        t0 = lax.rem(base, t_len)
        pltpu.sync_copy(idx_hbm.at[pl.ds(base, b_per_w)], idx_v)
        for c in range(n_chunks):
            gather = pltpu.async_copy(
                tok_hbm.at[idx_v.at[pl.ds(c * CH, CH)]], tok_v, sem)
            pltpu.sync_copy(pos_hbm.at[pl.ds(t0 + c * CH, CH)], pos_v)
            gather.wait()

            def add_row(r, _):
                def add_group(j, _):
                    v = pos_v[r, pl.ds(j * LANES, LANES)]
                    plsc.addupdate(tok_v.at[r, pl.ds(j * LANES, LANES)], v)
                    return 0
                return lax.fori_loop(0, D // LANES, add_group, 0)

            lax.fori_loop(0, CH, add_row, 0)
            pltpu.sync_copy(tok_v, out_hbm.at[pl.ds(base + c * CH, CH)])

    return k


@jax.jit
def kernel(x, tok_table, pos_table):
    b, t = x.shape
    idx = x.reshape(-1).astype(jnp.int32)
    out = _emb_kernel(b * t, t)(idx, tok_table, pos_table)
    return out.reshape(b, t, D)


# speedup vs baseline: 1.8740x; 1.8740x over previous
import functools

import jax
import jax.numpy as jnp
from jax import lax
from jax.experimental import pallas as pl
from jax.experimental.pallas import tpu as pltpu
from jax.experimental.pallas import tpu_sc as plsc

D = 1024
NC, NS = 2, 16
NW = NC * NS
LANES = 16
CH = 16


def _emb_kernel(b_sz, t_len):
    tw = t_len // NW
    n_ck = tw // CH
    n_total = b_sz * n_ck
    mesh = plsc.VectorSubcoreMesh(
        core_axis_name="c", subcore_axis_name="s", num_cores=NC,
        num_subcores=NS)

    @functools.partial(
        pl.kernel,
        out_type=jax.ShapeDtypeStruct((b_sz * t_len, D), jnp.float32),
        mesh=mesh,
        scratch_types=[
            pltpu.VMEM((b_sz * tw,), jnp.int32),
            pltpu.VMEM((tw, D), jnp.float32),
            pltpu.VMEM((CH, D), jnp.float32),
            pltpu.VMEM((CH, D), jnp.float32),
            pltpu.SemaphoreType.DMA,
            pltpu.SemaphoreType.DMA,
            pltpu.SemaphoreType.DMA,
            pltpu.SemaphoreType.DMA,
            pltpu.SemaphoreType.DMA,
        ],
    )
    def k(idx_hbm, tok_hbm, pos_hbm, out_hbm,
          idx_v, pos_v, tok0, tok1, sg0, sg1, so0, so1, sp):
        wid = lax.axis_index("s") * NC + lax.axis_index("c")
        tbase = wid * tw
        pos_cp = pltpu.async_copy(pos_hbm.at[pl.ds(tbase, tw)], pos_v, sp)
        for b in range(b_sz):
            pltpu.sync_copy(idx_hbm.at[pl.ds(b * t_len + tbase, tw)],
                            idx_v.at[pl.ds(b * tw, tw)])

        toks = (tok0, tok1)
        sgs = (sg0, sg1)
        sos = (so0, so1)

        def chunk_args(kk):
            b, c = kk // n_ck, kk % n_ck
            idx_sl = idx_v.at[pl.ds(b * tw + c * CH, CH)]
            out_off = b * t_len + tbase + c * CH
            return idx_sl, out_off, c * CH

        gathers = [None] * n_total
        outs = [None] * n_total
        idx_sl0, _, _ = chunk_args(0)
        gathers[0] = pltpu.async_copy(tok_hbm.at[idx_sl0], toks[0], sgs[0])
        pos_cp.wait()
        for kk in range(n_total):
            p = kk % 2
            cur = toks[p]
            if kk + 1 < n_total:
                if kk >= 1:
                    outs[kk - 1].wait()
                idx_sl, _, _ = chunk_args(kk + 1)
                gathers[kk + 1] = pltpu.async_copy(
                    tok_hbm.at[idx_sl], toks[1 - p], sgs[1 - p])
            gathers[kk].wait()
            _, out_off, prow = chunk_args(kk)

            def add_row(r, _):
                for j in range(D // LANES):
                    v = pos_v[prow + r, pl.ds(j * LANES, LANES)]
                    plsc.addupdate(cur.at[r, pl.ds(j * LANES, LANES)], v)
                return 0

            lax.fori_loop(0, CH, add_row, 0)
            outs[kk] = pltpu.async_copy(
                cur, out_hbm.at[pl.ds(out_off, CH)], sos[p])
        outs[n_total - 2].wait()
        outs[n_total - 1].wait()

    return k


@jax.jit
def kernel(x, tok_table, pos_table):
    b, t = x.shape
    idx = x.reshape(-1).astype(jnp.int32)
    out = _emb_kernel(b, t)(idx, tok_table, pos_table)
    return out.reshape(b, t, D)


# speedup vs baseline: 1.9312x; 1.0305x over previous
import functools

import jax
import jax.numpy as jnp
from jax import lax
from jax.experimental import pallas as pl
from jax.experimental.pallas import tpu as pltpu
from jax.experimental.pallas import tpu_sc as plsc

D = 1024
NC, NS = 2, 16
NW = NC * NS
LANES = 16
CH = 16
NBUF = 3


def _emb_kernel(b_sz, t_len):
    tw = t_len // NW
    n_ck = tw // CH
    n_total = b_sz * n_ck
    mesh = plsc.VectorSubcoreMesh(
        core_axis_name="c", subcore_axis_name="s", num_cores=NC,
        num_subcores=NS)

    @functools.partial(
        pl.kernel,
        out_type=jax.ShapeDtypeStruct((b_sz * t_len, D), jnp.float32),
        mesh=mesh,
        scratch_types=[
            pltpu.VMEM((b_sz * tw,), jnp.int32),
            pltpu.VMEM((tw, D), jnp.float32),
            pltpu.VMEM((CH, D), jnp.float32),
            pltpu.VMEM((CH, D), jnp.float32),
            pltpu.VMEM((CH, D), jnp.float32),
            pltpu.SemaphoreType.DMA,
            pltpu.SemaphoreType.DMA,
            pltpu.SemaphoreType.DMA,
            pltpu.SemaphoreType.DMA,
            pltpu.SemaphoreType.DMA,
            pltpu.SemaphoreType.DMA,
            pltpu.SemaphoreType.DMA,
        ],
    )
    def k(idx_hbm, tok_hbm, pos_hbm, out_hbm,
          idx_v, pos_v, tok0, tok1, tok2,
          sg0, sg1, sg2, so0, so1, so2, sp):
        wid = lax.axis_index("s") * NC + lax.axis_index("c")
        tbase = wid * tw
        pos_cp = pltpu.async_copy(pos_hbm.at[pl.ds(tbase, tw)], pos_v, sp)
        for b in range(b_sz):
            pltpu.sync_copy(idx_hbm.at[pl.ds(b * t_len + tbase, tw)],
                            idx_v.at[pl.ds(b * tw, tw)])

        toks = (tok0, tok1, tok2)
        sgs = (sg0, sg1, sg2)
        sos = (so0, so1, so2)

        def chunk_args(kk):
            b, c = kk // n_ck, kk % n_ck
            idx_sl = idx_v.at[pl.ds(b * tw + c * CH, CH)]
            out_off = b * t_len + tbase + c * CH
            return idx_sl, out_off, c * CH

        def start_gather(kk):
            idx_sl, _, _ = chunk_args(kk)
            return pltpu.async_copy(
                tok_hbm.at[idx_sl], toks[kk % NBUF], sgs[kk % NBUF])

        gathers = [None] * n_total
        outs = [None] * n_total
        gathers[0] = start_gather(0)
        gathers[1] = start_gather(1)
        pos_cp.wait()
        for kk in range(n_total):
            p = kk % NBUF
            cur = toks[p]
            if kk + 2 < n_total:
                if kk >= 1:
                    outs[kk - 1].wait()
                gathers[kk + 2] = start_gather(kk + 2)
            gathers[kk].wait()
            _, out_off, prow = chunk_args(kk)

            def add_row(r, _):
                for j in range(D // LANES):
                    v = pos_v[prow + r, pl.ds(j * LANES, LANES)]
                    plsc.addupdate(cur.at[r, pl.ds(j * LANES, LANES)], v)
                return 0

            lax.fori_loop(0, CH, add_row, 0)

            outs[kk] = pltpu.async_copy(
                cur, out_hbm.at[pl.ds(out_off, CH)], sos[p])
        outs[n_total - 2].wait()
        outs[n_total - 1].wait()

    return k


@jax.jit
def kernel(x, tok_table, pos_table):
    b, t = x.shape
    idx = x.reshape(-1).astype(jnp.int32)
    out = _emb_kernel(b, t)(idx, tok_table, pos_table)
    return out.reshape(b, t, D)


# speedup vs baseline: 3.6212x; 1.8751x over previous
import functools

import jax
import jax.numpy as jnp
from jax import lax
from jax.experimental import pallas as pl
from jax.experimental.pallas import tpu as pltpu
from jax.experimental.pallas import tpu_sc as plsc

D = 1024
NC, NS = 2, 16
NW = NC * NS
LANES = 16
CH = 8
NR = 3


def _emb_kernel(b_sz, t_len):
    tw = t_len // NW
    n_g = tw // CH
    mesh = plsc.VectorSubcoreMesh(
        core_axis_name="c", subcore_axis_name="s", num_cores=NC,
        num_subcores=NS)

    tok_scratch = [pltpu.VMEM((CH, D), jnp.float32)
                   for _ in range(NR * b_sz)]

    @functools.partial(
        pl.kernel,
        out_type=jax.ShapeDtypeStruct((b_sz * t_len, D), jnp.float32),
        mesh=mesh,
        scratch_types=[
            pltpu.VMEM((b_sz * tw,), jnp.int32),
            pltpu.VMEM((CH, D), jnp.float32),
            pltpu.VMEM((CH, D), jnp.float32),
            *tok_scratch,
            *([pltpu.SemaphoreType.DMA] * (2 * NR + 2)),
        ],
    )
    def k(idx_hbm, tok_hbm, pos_hbm, out_hbm,
          idx_v, pos0, pos1, *rest):
        toks = [rest[i * b_sz:(i + 1) * b_sz] for i in range(NR)]
        sg = rest[NR * b_sz:NR * b_sz + NR]
        so = rest[NR * b_sz + NR:NR * b_sz + 2 * NR]
        sp = rest[NR * b_sz + 2 * NR:NR * b_sz + 2 * NR + 2]
        poss = (pos0, pos1)

        wid = lax.axis_index("s") * NC + lax.axis_index("c")
        tbase = wid * tw
        for b in range(b_sz):
            pltpu.sync_copy(idx_hbm.at[pl.ds(b * t_len + tbase, tw)],
                            idx_v.at[pl.ds(b * tw, tw)])

        def start_pos(g):
            return pltpu.async_copy(
                pos_hbm.at[pl.ds(tbase + g * CH, CH)], poss[g % 2],
                sp[g % 2])

        def start_gathers(g):
            cps = []
            for b in range(b_sz):
                idx_sl = idx_v.at[pl.ds(b * tw + g * CH, CH)]
                cps.append(pltpu.async_copy(
                    tok_hbm.at[idx_sl], toks[g % NR][b], sg[g % NR]))
            return cps

        def start_outs(g):
            cps = []
            for b in range(b_sz):
                out_off = b * t_len + tbase + g * CH
                cps.append(pltpu.async_copy(
                    toks[g % NR][b], out_hbm.at[pl.ds(out_off, CH)],
                    so[g % NR]))
            return cps

        pos_cps = [None] * n_g
        g_cps = [None] * n_g
        o_cps = [None] * n_g
        drained = set()
        pos_cps[0] = start_pos(0)
        g_cps[0] = start_gathers(0)
        pos_cps[1] = start_pos(1)
        g_cps[1] = start_gathers(1)

        for g in range(n_g):
            for cp in g_cps[g]:
                cp.wait()
            pos_cps[g].wait()
            cur = toks[g % NR]
            pv = poss[g % 2]

            def add_row(r, _):
                for j in range(D // LANES):
                    v = pv[r, pl.ds(j * LANES, LANES)]
                    for b in range(b_sz):
                        plsc.addupdate(
                            cur[b].at[r, pl.ds(j * LANES, LANES)], v)
                return 0

            lax.fori_loop(0, CH, add_row, 0)
            o_cps[g] = start_outs(g)
            if g + 2 < n_g:
                if g >= 1:
                    for cp in o_cps[g - 1]:
                        cp.wait()
                    drained.add(g - 1)
                pos_cps[g + 2] = start_pos(g + 2)
                g_cps[g + 2] = start_gathers(g + 2)
        for g in range(n_g):
            if g not in drained:
                for cp in o_cps[g]:
                    cp.wait()

    return k


@jax.jit
def kernel(x, tok_table, pos_table):
    b, t = x.shape
    idx = x.reshape(-1).astype(jnp.int32)
    out = _emb_kernel(b, t)(idx, tok_table, pos_table)
    return out.reshape(b, t, D)
